# baseline (speedup 1.0000x reference)
import functools

import jax
import jax.numpy as jnp
from jax import lax
from jax.experimental import pallas as pl
from jax.experimental.pallas import tpu as pltpu
from jax.experimental.pallas import tpu_sc as plsc

N = 10000
IN_DIM = 128
OUT_DIM = 256
E = 320000

NPAD = 10240
CHUNK = 128
NC, NS = 2, 16
NW = NC * NS
NCHT = 2560
CPT = NCHT // NW
CPT0 = 160
PHASE = 32
ROWS_PER_TILE = NPAD // NS

_MESH = plsc.VectorSubcoreMesh(core_axis_name="c", subcore_axis_name="s")


def _fill(vmem_ref, value):
    v = jnp.full((16,), value, jnp.float32)

    @pl.loop(0, vmem_ref.shape[0])
    def _(r):
        @pl.loop(0, 8)
        def _(c):
            vmem_ref[r, pl.ds(c * 16, 16)] = v


def _sc_degree(dst_c):
    @functools.partial(
        pl.kernel,
        mesh=_MESH,
        out_type=jax.ShapeDtypeStruct((NC, NPAD, IN_DIM), jnp.float32),
        scratch_types=[
            pltpu.VMEM((CPT, CHUNK), jnp.int32),
            pltpu.VMEM((CHUNK, IN_DIM), jnp.float32),
            pltpu.VMEM((64, IN_DIM), jnp.float32),
            pltpu.VMEM_SHARED((NPAD, IN_DIM), jnp.float32),
            pltpu.SemaphoreType.DMA,
        ],
    )
    def k(dst_hbm, out_hbm, idx_v, ones_v, zeros_v, acc_sh, sem):
        cid = lax.axis_index("c")
        sid = lax.axis_index("s")
        wid = sid * NC + cid

        _fill(ones_v, 1.0)
        _fill(zeros_v, 0.0)

        @pl.loop(0, ROWS_PER_TILE // 64)
        def _(t):
            pltpu.sync_copy(
                zeros_v, acc_sh.at[pl.ds(sid * ROWS_PER_TILE + t * 64, 64)])

        plsc.subcore_barrier()

        pltpu.sync_copy(dst_hbm.at[pl.ds(wid * CPT, CPT)], idx_v)

        @pl.loop(0, CPT)
        def _(j):
            pltpu.sync_copy(ones_v, acc_sh.at[idx_v.at[j]], add=True)

        plsc.subcore_barrier()
        rows = pl.ds(sid * ROWS_PER_TILE, ROWS_PER_TILE)
        pltpu.sync_copy(acc_sh.at[rows], out_hbm.at[cid, rows])

    return k(dst_c)


def _sc_aggregate(xs, src_c, dst_c):
    @functools.partial(
        pl.kernel,
        mesh=_MESH,
        out_type=jax.ShapeDtypeStruct((NPAD, IN_DIM), jnp.float32),
        scratch_types=[
            pltpu.VMEM((PHASE, CHUNK), jnp.int32),
            pltpu.VMEM((PHASE, CHUNK), jnp.int32),
            pltpu.VMEM((CHUNK, IN_DIM), jnp.float32),
            pltpu.VMEM((CHUNK, IN_DIM), jnp.float32),
            pltpu.VMEM((16, IN_DIM), jnp.float32),
            pltpu.VMEM_SHARED((NPAD, IN_DIM), jnp.float32),
            pltpu.SemaphoreType.DMA,
            pltpu.SemaphoreType.DMA,
            pltpu.SemaphoreType.DMA,
        ],
    )
    def k(xs_hbm, src_hbm, dst_hbm, out_hbm, src_v, dst_v, b0, b1, zeros_v,
          acc_sh, sg0, sg1, ss):
        cid = lax.axis_index("c")
        sid = lax.axis_index("s")

        def pipeline(base_chunk, cpt):
            half = PHASE

            @pl.loop(0, cpt // PHASE)
            def _(h):
                base = base_chunk + h * half
                pltpu.sync_copy(src_hbm.at[pl.ds(base, half)], src_v)
                pltpu.sync_copy(dst_hbm.at[pl.ds(base, half)], dst_v)

                pltpu.async_copy(xs_hbm.at[src_v.at[0]], b0, sg0)
                pltpu.async_copy(xs_hbm.at[src_v.at[1]], b1, sg1)

                @pl.loop(0, half // 2)
                def _(t):
                    j0 = 2 * t
                    j1 = j0 + 1
                    pltpu.make_async_copy(
                        xs_hbm.at[src_v.at[j0]], b0, sg0).wait()
                    pltpu.async_copy(b0, acc_sh.at[dst_v.at[j0]], ss, add=True)
                    pltpu.make_async_copy(
                        xs_hbm.at[src_v.at[j1]], b1, sg1).wait()
                    pltpu.async_copy(b1, acc_sh.at[dst_v.at[j1]], ss, add=True)
                    pltpu.make_async_copy(b0, acc_sh.at[dst_v.at[j0]], ss).wait()
                    pltpu.make_async_copy(b1, acc_sh.at[dst_v.at[j1]], ss).wait()

                    @pl.when(j0 + 2 < half)
                    def _():
                        pltpu.async_copy(xs_hbm.at[src_v.at[j0 + 2]], b0, sg0)
                        pltpu.async_copy(xs_hbm.at[src_v.at[j1 + 2]], b1, sg1)

        @pl.when(cid == 0)
        def _():
            _fill(zeros_v, 0.0)

            @pl.loop(0, ROWS_PER_TILE // 16)
            def _(t):
                pltpu.sync_copy(
                    zeros_v,
                    acc_sh.at[pl.ds(sid * ROWS_PER_TILE + t * 16, 16)])

            plsc.subcore_barrier()
            pipeline(sid * CPT0, CPT0)
            plsc.subcore_barrier()
            rows = pl.ds(sid * ROWS_PER_TILE, ROWS_PER_TILE)
            pltpu.sync_copy(acc_sh.at[rows], out_hbm.at[rows])

    return k(xs, src_c, dst_c)


_BR = 400


def _tc_scale(x, degp):
    def body(x_ref, d_ref, xs_ref):
        deg = d_ref[0, :, :1] + d_ref[1, :, :1] + 1.0
        xs_ref[...] = x_ref[...] * lax.rsqrt(deg)

    return pl.pallas_call(
        body,
        grid=(N // _BR,),
        in_specs=[
            pl.BlockSpec((_BR, IN_DIM), lambda i: (i, 0)),
            pl.BlockSpec((NC, _BR, IN_DIM), lambda i: (0, i, 0)),
        ],
        out_specs=pl.BlockSpec((_BR, IN_DIM), lambda i: (i, 0)),
        out_shape=jax.ShapeDtypeStruct((N, IN_DIM), jnp.float32),
    )(x, degp)


def _tc_final(accp, xs, degp, W, b2):
    def body(a_ref, xs_ref, d_ref, w_ref, b_ref, o_ref):
        deg = d_ref[0, :, :1] + d_ref[1, :, :1] + 1.0
        s = (a_ref[...] + xs_ref[...]) * lax.rsqrt(deg)
        o_ref[...] = (
            jnp.dot(s, w_ref[...], preferred_element_type=jnp.float32,
                    precision=lax.Precision.HIGHEST)
            + b_ref[...])

    return pl.pallas_call(
        body,
        grid=(N // _BR,),
        in_specs=[
            pl.BlockSpec((_BR, IN_DIM), lambda i: (i, 0)),
            pl.BlockSpec((_BR, IN_DIM), lambda i: (i, 0)),
            pl.BlockSpec((NC, _BR, IN_DIM), lambda i: (0, i, 0)),
            pl.BlockSpec((IN_DIM, OUT_DIM), lambda i: (0, 0)),
            pl.BlockSpec((1, OUT_DIM), lambda i: (0, 0)),
        ],
        out_specs=pl.BlockSpec((_BR, OUT_DIM), lambda i: (i, 0)),
        out_shape=jax.ShapeDtypeStruct((N, OUT_DIM), jnp.float32),
    )(accp, xs, degp, W, b2)


def kernel(x, edge_index, W, b):
    src = edge_index[0].astype(jnp.int32)
    dst = edge_index[1].astype(jnp.int32)
    pad_e = NCHT * CHUNK - E
    src_c = jnp.concatenate(
        [src, jnp.zeros((pad_e,), jnp.int32)]).reshape(NCHT, CHUNK)
    dst_c = jnp.concatenate(
        [dst, jnp.full((pad_e,), N, jnp.int32)]).reshape(NCHT, CHUNK)

    degp = _sc_degree(dst_c)
    xs = _tc_scale(x, degp)
    accp = _sc_aggregate(xs, src_c, dst_c)
    return _tc_final(accp, xs, degp, W, b.reshape(1, OUT_DIM))


# speedup vs baseline: 1.4844x; 1.4844x over previous
import functools

import jax
import jax.numpy as jnp
from jax import lax
from jax.experimental import pallas as pl
from jax.experimental.pallas import tpu as pltpu
from jax.experimental.pallas import tpu_sc as plsc

N = 10000
IN_DIM = 128
OUT_DIM = 256
E = 320000

NPAD = 10240
CHUNK = 128
NC, NS = 2, 16
NW = NC * NS
NCHT = 2560
CPT = NCHT // NW
CPT0 = 144
CPT1 = 16
PHASE = 16
ROWS_PER_TILE = NPAD // NS

_MESH = plsc.VectorSubcoreMesh(core_axis_name="c", subcore_axis_name="s")


def _fill(vmem_ref, value):
    v = jnp.full((16,), value, jnp.float32)

    @pl.loop(0, vmem_ref.shape[0])
    def _(r):
        @pl.loop(0, 8)
        def _(c):
            vmem_ref[r, pl.ds(c * 16, 16)] = v


def _sc_degree(dst_c):
    @functools.partial(
        pl.kernel,
        mesh=_MESH,
        out_type=jax.ShapeDtypeStruct((NC, NPAD, IN_DIM), jnp.float32),
        scratch_types=[
            pltpu.VMEM((CPT, CHUNK), jnp.int32),
            pltpu.VMEM((CHUNK, IN_DIM), jnp.float32),
            pltpu.VMEM((64, IN_DIM), jnp.float32),
            pltpu.VMEM_SHARED((NPAD, IN_DIM), jnp.float32),
            pltpu.SemaphoreType.DMA,
        ],
    )
    def k(dst_hbm, out_hbm, idx_v, ones_v, zeros_v, acc_sh, sem):
        cid = lax.axis_index("c")
        sid = lax.axis_index("s")
        wid = sid * NC + cid

        _fill(ones_v, 1.0)
        _fill(zeros_v, 0.0)

        @pl.loop(0, ROWS_PER_TILE // 64)
        def _(t):
            pltpu.sync_copy(
                zeros_v, acc_sh.at[pl.ds(sid * ROWS_PER_TILE + t * 64, 64)])

        plsc.subcore_barrier()

        pltpu.sync_copy(dst_hbm.at[pl.ds(wid * CPT, CPT)], idx_v)

        @pl.loop(0, CPT)
        def _(j):
            pltpu.sync_copy(ones_v, acc_sh.at[idx_v.at[j]], add=True)

        plsc.subcore_barrier()
        rows = pl.ds(sid * ROWS_PER_TILE, ROWS_PER_TILE)
        pltpu.sync_copy(acc_sh.at[rows], out_hbm.at[cid, rows])

    return k(dst_c)


def _sc_aggregate(xs, src_c, dst_c):
    @functools.partial(
        pl.kernel,
        mesh=_MESH,
        out_type=jax.ShapeDtypeStruct((NC, NPAD, IN_DIM), jnp.float32),
        scratch_types=[
            pltpu.VMEM((PHASE, CHUNK), jnp.int32),
            pltpu.VMEM((PHASE, CHUNK), jnp.int32),
            pltpu.VMEM((CHUNK, IN_DIM), jnp.float32),
            pltpu.VMEM((CHUNK, IN_DIM), jnp.float32),
            pltpu.VMEM((16, IN_DIM), jnp.float32),
            pltpu.VMEM_SHARED((NPAD, IN_DIM), jnp.float32),
            pltpu.SemaphoreType.DMA,
            pltpu.SemaphoreType.DMA,
            pltpu.SemaphoreType.DMA,
        ],
    )
    def k(xs_hbm, src_hbm, dst_hbm, out_hbm, src_v, dst_v, b0, b1, zeros_v,
          acc_sh, sg0, sg1, ss):
        cid = lax.axis_index("c")
        sid = lax.axis_index("s")

        def pipeline(base_chunk, cpt):
            half = PHASE

            @pl.loop(0, cpt // PHASE)
            def _(h):
                base = base_chunk + h * half
                pltpu.sync_copy(src_hbm.at[pl.ds(base, half)], src_v)
                pltpu.sync_copy(dst_hbm.at[pl.ds(base, half)], dst_v)

                pltpu.async_copy(xs_hbm.at[src_v.at[0]], b0, sg0)
                pltpu.async_copy(xs_hbm.at[src_v.at[1]], b1, sg1)

                @pl.loop(0, half // 2)
                def _(t):
                    j0 = 2 * t
                    j1 = j0 + 1
                    pltpu.make_async_copy(
                        xs_hbm.at[src_v.at[j0]], b0, sg0).wait()
                    pltpu.async_copy(b0, acc_sh.at[dst_v.at[j0]], ss, add=True)
                    pltpu.make_async_copy(
                        xs_hbm.at[src_v.at[j1]], b1, sg1).wait()
                    pltpu.async_copy(b1, acc_sh.at[dst_v.at[j1]], ss, add=True)
                    pltpu.make_async_copy(b0, acc_sh.at[dst_v.at[j0]], ss).wait()
                    pltpu.make_async_copy(b1, acc_sh.at[dst_v.at[j1]], ss).wait()

                    @pl.when(j0 + 2 < half)
                    def _():
                        pltpu.async_copy(xs_hbm.at[src_v.at[j0 + 2]], b0, sg0)
                        pltpu.async_copy(xs_hbm.at[src_v.at[j1 + 2]], b1, sg1)

        _fill(zeros_v, 0.0)

        @pl.loop(0, ROWS_PER_TILE // 16)
        def _(t):
            pltpu.sync_copy(
                zeros_v, acc_sh.at[pl.ds(sid * ROWS_PER_TILE + t * 16, 16)])

        plsc.subcore_barrier()

        @pl.when(cid == 0)
        def _():
            pipeline(sid * CPT0, CPT0)

        @pl.when(cid == 1)
        def _():
            pipeline(NS * CPT0 + sid * CPT1, CPT1)

        plsc.subcore_barrier()
        rows = pl.ds(sid * ROWS_PER_TILE, ROWS_PER_TILE)
        pltpu.sync_copy(acc_sh.at[rows], out_hbm.at[cid, rows])

    return k(xs, src_c, dst_c)


_BR = 400


def _tc_scale(x, degp):
    def body(x_ref, d_ref, xs_ref):
        deg = d_ref[0, :, :1] + d_ref[1, :, :1] + 1.0
        xs_ref[...] = x_ref[...] * lax.rsqrt(deg)

    return pl.pallas_call(
        body,
        grid=(N // _BR,),
        in_specs=[
            pl.BlockSpec((_BR, IN_DIM), lambda i: (i, 0)),
            pl.BlockSpec((NC, _BR, IN_DIM), lambda i: (0, i, 0)),
        ],
        out_specs=pl.BlockSpec((_BR, IN_DIM), lambda i: (i, 0)),
        out_shape=jax.ShapeDtypeStruct((N, IN_DIM), jnp.float32),
    )(x, degp)


def _tc_final(accp, xs, degp, W, b2):
    def body(a_ref, xs_ref, d_ref, w_ref, b_ref, o_ref):
        deg = d_ref[0, :, :1] + d_ref[1, :, :1] + 1.0
        s = (a_ref[0] + a_ref[1] + xs_ref[...]) * lax.rsqrt(deg)
        o_ref[...] = (
            jnp.dot(s, w_ref[...], preferred_element_type=jnp.float32,
                    precision=lax.Precision.HIGHEST)
            + b_ref[...])

    return pl.pallas_call(
        body,
        grid=(N // _BR,),
        in_specs=[
            pl.BlockSpec((NC, _BR, IN_DIM), lambda i: (0, i, 0)),
            pl.BlockSpec((_BR, IN_DIM), lambda i: (i, 0)),
            pl.BlockSpec((NC, _BR, IN_DIM), lambda i: (0, i, 0)),
            pl.BlockSpec((IN_DIM, OUT_DIM), lambda i: (0, 0)),
            pl.BlockSpec((1, OUT_DIM), lambda i: (0, 0)),
        ],
        out_specs=pl.BlockSpec((_BR, OUT_DIM), lambda i: (i, 0)),
        out_shape=jax.ShapeDtypeStruct((N, OUT_DIM), jnp.float32),
    )(accp, xs, degp, W, b2)


def kernel(x, edge_index, W, b):
    src = edge_index[0].astype(jnp.int32)
    dst = edge_index[1].astype(jnp.int32)
    pad_e = NCHT * CHUNK - E
    src_c = jnp.concatenate(
        [src, jnp.zeros((pad_e,), jnp.int32)]).reshape(NCHT, CHUNK)
    dst_c = jnp.concatenate(
        [dst, jnp.full((pad_e,), N, jnp.int32)]).reshape(NCHT, CHUNK)

    degp = _sc_degree(dst_c)
    xs = _tc_scale(x, degp)
    accp = _sc_aggregate(xs, src_c, dst_c)
    return _tc_final(accp, xs, degp, W, b.reshape(1, OUT_DIM))


# speedup vs baseline: 1.5012x; 1.0113x over previous
import functools

import jax
import jax.numpy as jnp
from jax import lax
from jax.experimental import pallas as pl
from jax.experimental.pallas import tpu as pltpu
from jax.experimental.pallas import tpu_sc as plsc

N = 10000
IN_DIM = 128
OUT_DIM = 256
E = 320000

NPAD = 10240
CHUNK = 128
NC, NS = 2, 16
NW = NC * NS
NCHT = 2560
CPT = NCHT // NW
CPT0 = 152
CPT1 = 8
PHASE = 8
ROWS_PER_TILE = NPAD // NS

_MESH = plsc.VectorSubcoreMesh(core_axis_name="c", subcore_axis_name="s")


def _fill(vmem_ref, value):
    v = jnp.full((16,), value, jnp.float32)

    @pl.loop(0, vmem_ref.shape[0])
    def _(r):
        @pl.loop(0, 8)
        def _(c):
            vmem_ref[r, pl.ds(c * 16, 16)] = v


def _sc_degree(dst_c):
    @functools.partial(
        pl.kernel,
        mesh=_MESH,
        out_type=jax.ShapeDtypeStruct((NC, NPAD, IN_DIM), jnp.float32),
        scratch_types=[
            pltpu.VMEM((CPT, CHUNK), jnp.int32),
            pltpu.VMEM((CHUNK, IN_DIM), jnp.float32),
            pltpu.VMEM((64, IN_DIM), jnp.float32),
            pltpu.VMEM_SHARED((NPAD, IN_DIM), jnp.float32),
            pltpu.SemaphoreType.DMA,
        ],
    )
    def k(dst_hbm, out_hbm, idx_v, ones_v, zeros_v, acc_sh, sem):
        cid = lax.axis_index("c")
        sid = lax.axis_index("s")
        wid = sid * NC + cid

        _fill(ones_v, 1.0)
        _fill(zeros_v, 0.0)

        @pl.loop(0, ROWS_PER_TILE // 64)
        def _(t):
            pltpu.sync_copy(
                zeros_v, acc_sh.at[pl.ds(sid * ROWS_PER_TILE + t * 64, 64)])

        plsc.subcore_barrier()

        pltpu.sync_copy(dst_hbm.at[pl.ds(wid * CPT, CPT)], idx_v)

        @pl.loop(0, CPT)
        def _(j):
            pltpu.sync_copy(ones_v, acc_sh.at[idx_v.at[j]], add=True)

        plsc.subcore_barrier()
        rows = pl.ds(sid * ROWS_PER_TILE, ROWS_PER_TILE)
        pltpu.sync_copy(acc_sh.at[rows], out_hbm.at[cid, rows])

    return k(dst_c)


def _sc_aggregate(xs, src_c, dst_c):
    @functools.partial(
        pl.kernel,
        mesh=_MESH,
        out_type=jax.ShapeDtypeStruct((NC, NPAD, IN_DIM), jnp.float32),
        scratch_types=[
            pltpu.VMEM((PHASE, CHUNK), jnp.int32),
            pltpu.VMEM((PHASE, CHUNK), jnp.int32),
            pltpu.VMEM((CHUNK, IN_DIM), jnp.float32),
            pltpu.VMEM((CHUNK, IN_DIM), jnp.float32),
            pltpu.VMEM((16, IN_DIM), jnp.float32),
            pltpu.VMEM_SHARED((NPAD, IN_DIM), jnp.float32),
            pltpu.SemaphoreType.DMA,
            pltpu.SemaphoreType.DMA,
            pltpu.SemaphoreType.DMA,
        ],
    )
    def k(xs_hbm, src_hbm, dst_hbm, out_hbm, src_v, dst_v, b0, b1, zeros_v,
          acc_sh, sg0, sg1, ss):
        cid = lax.axis_index("c")
        sid = lax.axis_index("s")

        def pipeline(base_chunk, cpt):
            half = PHASE

            @pl.loop(0, cpt // PHASE)
            def _(h):
                base = base_chunk + h * half
                pltpu.sync_copy(src_hbm.at[pl.ds(base, half)], src_v)
                pltpu.sync_copy(dst_hbm.at[pl.ds(base, half)], dst_v)

                pltpu.async_copy(xs_hbm.at[src_v.at[0]], b0, sg0)
                pltpu.async_copy(xs_hbm.at[src_v.at[1]], b1, sg1)

                @pl.loop(0, half // 2)
                def _(t):
                    j0 = 2 * t
                    j1 = j0 + 1
                    pltpu.make_async_copy(
                        xs_hbm.at[src_v.at[j0]], b0, sg0).wait()
                    pltpu.async_copy(b0, acc_sh.at[dst_v.at[j0]], ss, add=True)
                    pltpu.make_async_copy(
                        xs_hbm.at[src_v.at[j1]], b1, sg1).wait()
                    pltpu.async_copy(b1, acc_sh.at[dst_v.at[j1]], ss, add=True)
                    pltpu.make_async_copy(b0, acc_sh.at[dst_v.at[j0]], ss).wait()
                    pltpu.make_async_copy(b1, acc_sh.at[dst_v.at[j1]], ss).wait()

                    @pl.when(j0 + 2 < half)
                    def _():
                        pltpu.async_copy(xs_hbm.at[src_v.at[j0 + 2]], b0, sg0)
                        pltpu.async_copy(xs_hbm.at[src_v.at[j1 + 2]], b1, sg1)

        _fill(zeros_v, 0.0)

        @pl.loop(0, ROWS_PER_TILE // 16)
        def _(t):
            pltpu.sync_copy(
                zeros_v, acc_sh.at[pl.ds(sid * ROWS_PER_TILE + t * 16, 16)])

        plsc.subcore_barrier()

        @pl.when(cid == 0)
        def _():
            pipeline(sid * CPT0, CPT0)

        @pl.when(cid == 1)
        def _():
            pipeline(NS * CPT0 + sid * CPT1, CPT1)

        plsc.subcore_barrier()
        rows = pl.ds(sid * ROWS_PER_TILE, ROWS_PER_TILE)
        pltpu.sync_copy(acc_sh.at[rows], out_hbm.at[cid, rows])

    return k(xs, src_c, dst_c)


_BR = 400


def _tc_scale(x, degp):
    def body(x_ref, d_ref, xs_ref):
        deg = d_ref[0, :, :1] + d_ref[1, :, :1] + 1.0
        xs_ref[...] = x_ref[...] * lax.rsqrt(deg)

    return pl.pallas_call(
        body,
        grid=(N // _BR,),
        in_specs=[
            pl.BlockSpec((_BR, IN_DIM), lambda i: (i, 0)),
            pl.BlockSpec((NC, _BR, IN_DIM), lambda i: (0, i, 0)),
        ],
        out_specs=pl.BlockSpec((_BR, IN_DIM), lambda i: (i, 0)),
        out_shape=jax.ShapeDtypeStruct((N, IN_DIM), jnp.float32),
    )(x, degp)


def _tc_final(accp, xs, degp, W, b2):
    def body(a_ref, xs_ref, d_ref, w_ref, b_ref, o_ref):
        deg = d_ref[0, :, :1] + d_ref[1, :, :1] + 1.0
        s = (a_ref[0] + a_ref[1] + xs_ref[...]) * lax.rsqrt(deg)
        o_ref[...] = (
            jnp.dot(s, w_ref[...], preferred_element_type=jnp.float32,
                    precision=lax.Precision.HIGHEST)
            + b_ref[...])

    return pl.pallas_call(
        body,
        grid=(N // _BR,),
        in_specs=[
            pl.BlockSpec((NC, _BR, IN_DIM), lambda i: (0, i, 0)),
            pl.BlockSpec((_BR, IN_DIM), lambda i: (i, 0)),
            pl.BlockSpec((NC, _BR, IN_DIM), lambda i: (0, i, 0)),
            pl.BlockSpec((IN_DIM, OUT_DIM), lambda i: (0, 0)),
            pl.BlockSpec((1, OUT_DIM), lambda i: (0, 0)),
        ],
        out_specs=pl.BlockSpec((_BR, OUT_DIM), lambda i: (i, 0)),
        out_shape=jax.ShapeDtypeStruct((N, OUT_DIM), jnp.float32),
    )(accp, xs, degp, W, b2)


def kernel(x, edge_index, W, b):
    src = edge_index[0].astype(jnp.int32)
    dst = edge_index[1].astype(jnp.int32)
    pad_e = NCHT * CHUNK - E
    src_c = jnp.concatenate(
        [src, jnp.zeros((pad_e,), jnp.int32)]).reshape(NCHT, CHUNK)
    dst_c = jnp.concatenate(
        [dst, jnp.full((pad_e,), N, jnp.int32)]).reshape(NCHT, CHUNK)

    degp = _sc_degree(dst_c)
    xs = _tc_scale(x, degp)
    accp = _sc_aggregate(xs, src_c, dst_c)
    return _tc_final(accp, xs, degp, W, b.reshape(1, OUT_DIM))
